# baseline (speedup 1.0000x reference)
import functools
import math

import jax
import jax.numpy as jnp
from jax.experimental import pallas as pl
from jax.experimental.pallas import tpu as pltpu


def _round_up(x: int, m: int) -> int:
    return ((x + m - 1) // m) * m


def _interp_matrix(out_size: int, in_size: int) -> jnp.ndarray:
    if in_size == 1:
        return jnp.ones((out_size, 1), jnp.float32)
    if out_size == 1:
        return jnp.zeros((1, in_size), jnp.float32).at[0, 0].set(1.0)
    dst = jnp.arange(out_size, dtype=jnp.float32)
    src = dst * ((in_size - 1) / (out_size - 1))
    i0 = jnp.clip(jnp.floor(src).astype(jnp.int32), 0, in_size - 1)
    i1 = jnp.clip(i0 + 1, 0, in_size - 1)
    frac = src - i0.astype(jnp.float32)
    rows = jnp.arange(out_size)
    m = jnp.zeros((out_size, in_size), jnp.float32)
    m = m.at[rows, i0].add(1.0 - frac)
    m = m.at[rows, i1].add(frac)
    return m


def _upsample_kernel(c_lp, h, low_ref, mh_ref, mwt_ref, o_ref):
    lw = low_ref[0].astype(jnp.bfloat16)
    wi = jnp.dot(lw, mwt_ref[...], preferred_element_type=jnp.float32)
    wib = wi.astype(jnp.bfloat16)
    mh = mh_ref[...]
    for c in range(c_lp):
        zc = jnp.dot(mh, wib[c * h:(c + 1) * h],
                     preferred_element_type=jnp.float32)
        o_ref[0, c] = zc.astype(jnp.bfloat16)


def _conv_kernel(c_hp, high_ref, lowup_ref, w_ref, b_ref, o_ref, kbuf):
    kbuf[0:c_hp, :] = high_ref[0].astype(jnp.bfloat16)
    kbuf[c_hp:, :] = lowup_ref[0]
    acc = jnp.dot(w_ref[...], kbuf[...], preferred_element_type=jnp.float32)
    acc = acc + b_ref[...]
    acc = acc * jnp.clip(acc + 3.0, 0.0, 6.0) * (1.0 / 6.0)
    o_ref[0] = acc.astype(o_ref.dtype)


def kernel(low_res, high_res, weight, bias):
    n, c_lp, h, w = low_res.shape
    n2, c_hp, H, W = high_res.shape
    assert n == n2
    c_out = weight.shape[0]
    c_in = c_lp + c_hp
    out_dtype = high_res.dtype

    r_align = max(8, 128 // math.gcd(W, 128))
    R = r_align
    while R * 2 <= H and R * W < 8192:
        R *= 2
    if R > H:
        R = _round_up(H, r_align)
    Hp = _round_up(H, R)
    T = R * W

    m_h = _interp_matrix(H, h)
    m_wt = _interp_matrix(W, w).T
    if Hp > H:
        m_h = jnp.pad(m_h, ((0, Hp - H), (0, 0)))
        high_res = jnp.pad(high_res, ((0, 0), (0, 0), (0, Hp - H), (0, 0)))

    mh_b = m_h.astype(jnp.bfloat16)
    mwt_b = m_wt.astype(jnp.bfloat16)
    low2d = low_res.reshape(n, c_lp * h, w)

    vmem_limit = 48 * 1024 * 1024

    low_up = pl.pallas_call(
        functools.partial(_upsample_kernel, c_lp, h),
        out_shape=jax.ShapeDtypeStruct((n, c_lp, Hp, W), jnp.bfloat16),
        grid=(n,),
        in_specs=[
            pl.BlockSpec((1, c_lp * h, w), lambda i: (i, 0, 0)),
            pl.BlockSpec((Hp, h), lambda i: (0, 0)),
            pl.BlockSpec((w, W), lambda i: (0, 0)),
        ],
        out_specs=pl.BlockSpec((1, c_lp, Hp, W), lambda i: (i, 0, 0, 0)),
        compiler_params=pltpu.CompilerParams(
            dimension_semantics=("parallel",),
            vmem_limit_bytes=vmem_limit),
    )(low2d, mh_b, mwt_b)

    low_up_flat = low_up.reshape(n, c_lp, Hp * W)
    high_flat = high_res.reshape(n, c_hp, Hp * W)

    w_all = jnp.concatenate([weight[:, c_lp:], weight[:, :c_lp]],
                            axis=1).astype(jnp.bfloat16)
    b2d = bias.reshape(c_out, 1).astype(jnp.float32)

    out_flat = pl.pallas_call(
        functools.partial(_conv_kernel, c_hp),
        out_shape=jax.ShapeDtypeStruct((n, c_out, Hp * W), out_dtype),
        grid=(n, Hp // R),
        in_specs=[
            pl.BlockSpec((1, c_hp, T), lambda i, s: (i, 0, s)),
            pl.BlockSpec((1, c_lp, T), lambda i, s: (i, 0, s)),
            pl.BlockSpec((c_out, c_in), lambda i, s: (0, 0)),
            pl.BlockSpec((c_out, 1), lambda i, s: (0, 0)),
        ],
        out_specs=pl.BlockSpec((1, c_out, T), lambda i, s: (i, 0, s)),
        scratch_shapes=[pltpu.VMEM((c_in, T), jnp.bfloat16)],
        compiler_params=pltpu.CompilerParams(
            dimension_semantics=("parallel", "parallel"),
            vmem_limit_bytes=vmem_limit),
    )(high_flat, low_up_flat, w_all, b2d)

    out = out_flat.reshape(n, c_out, Hp, W)
    return out[:, :, :H, :] if Hp > H else out


# speedup vs baseline: 1.1094x; 1.1094x over previous
import functools
import math

import jax
import jax.numpy as jnp
import numpy as np
from jax.experimental import pallas as pl
from jax.experimental.pallas import tpu as pltpu


def _round_up(x: int, m: int) -> int:
    return ((x + m - 1) // m) * m


def _interp_matrix(out_size: int, in_size: int) -> np.ndarray:
    if in_size == 1:
        return np.ones((out_size, 1), np.float32)
    if out_size == 1:
        m = np.zeros((1, in_size), np.float32)
        m[0, 0] = 1.0
        return m
    dst = np.arange(out_size, dtype=np.float32)
    src = dst * np.float32((in_size - 1) / (out_size - 1))
    i0 = np.clip(np.floor(src).astype(np.int64), 0, in_size - 1)
    i1 = np.clip(i0 + 1, 0, in_size - 1)
    frac = (src - i0).astype(np.float32)
    rows = np.arange(out_size)
    m = np.zeros((out_size, in_size), np.float32)
    np.add.at(m, (rows, i0), 1.0 - frac)
    np.add.at(m, (rows, i1), frac)
    return m


def _upsample_kernel(c_lp, h, low_ref, mh_ref, mwt_ref, o_ref):
    lw = low_ref[0].astype(jnp.bfloat16)
    wi = jnp.dot(lw, mwt_ref[...], preferred_element_type=jnp.float32)
    wib = wi.astype(jnp.bfloat16)
    mh = mh_ref[...]
    for c in range(c_lp):
        zc = jnp.dot(mh, wib[c * h:(c + 1) * h],
                     preferred_element_type=jnp.float32)
        o_ref[0, c] = zc.astype(jnp.bfloat16)


def _conv_kernel(c_hp, high_ref, lowup_ref, w_ref, b_ref, o_ref, kbuf):
    kbuf[0:c_hp, :] = high_ref[0].astype(jnp.bfloat16)
    kbuf[c_hp:, :] = lowup_ref[0]
    acc = jnp.dot(w_ref[...], kbuf[...], preferred_element_type=jnp.float32)
    acc = acc + b_ref[...]
    acc = acc * jnp.clip(acc + 3.0, 0.0, 6.0) * (1.0 / 6.0)
    o_ref[0] = acc.astype(o_ref.dtype)


def kernel(low_res, high_res, weight, bias):
    n, c_lp, h, w = low_res.shape
    n2, c_hp, H, W = high_res.shape
    assert n == n2
    c_out = weight.shape[0]
    c_in = c_lp + c_hp
    out_dtype = high_res.dtype

    r_align = max(8, 128 // math.gcd(W, 128))
    R = r_align
    while R * 2 <= H and R * W < 8192:
        R *= 2
    if R > H:
        R = _round_up(H, r_align)
    Hp = _round_up(H, R)
    T = R * W

    m_h = _interp_matrix(H, h)
    m_wt = _interp_matrix(W, w).T
    if Hp > H:
        m_h = np.pad(m_h, ((0, Hp - H), (0, 0)))
        high_res = jnp.pad(high_res, ((0, 0), (0, 0), (0, Hp - H), (0, 0)))

    mh_b = m_h.astype(jnp.bfloat16)
    mwt_b = m_wt.astype(jnp.bfloat16)
    low2d = low_res.reshape(n, c_lp * h, w)

    vmem_limit = 48 * 1024 * 1024

    low_up = pl.pallas_call(
        functools.partial(_upsample_kernel, c_lp, h),
        out_shape=jax.ShapeDtypeStruct((n, c_lp, Hp, W), jnp.bfloat16),
        grid=(n,),
        in_specs=[
            pl.BlockSpec((1, c_lp * h, w), lambda i: (i, 0, 0)),
            pl.BlockSpec((Hp, h), lambda i: (0, 0)),
            pl.BlockSpec((w, W), lambda i: (0, 0)),
        ],
        out_specs=pl.BlockSpec((1, c_lp, Hp, W), lambda i: (i, 0, 0, 0)),
        compiler_params=pltpu.CompilerParams(
            dimension_semantics=("parallel",),
            vmem_limit_bytes=vmem_limit),
    )(low2d, mh_b, mwt_b)

    low_up_flat = low_up.reshape(n, c_lp, Hp * W)
    high_flat = high_res.reshape(n, c_hp, Hp * W)

    w_all = jnp.concatenate([weight[:, c_lp:], weight[:, :c_lp]],
                            axis=1).astype(jnp.bfloat16)
    b2d = bias.reshape(c_out, 1).astype(jnp.float32)

    out_flat = pl.pallas_call(
        functools.partial(_conv_kernel, c_hp),
        out_shape=jax.ShapeDtypeStruct((n, c_out, Hp * W), out_dtype),
        grid=(n, Hp // R),
        in_specs=[
            pl.BlockSpec((1, c_hp, T), lambda i, s: (i, 0, s)),
            pl.BlockSpec((1, c_lp, T), lambda i, s: (i, 0, s)),
            pl.BlockSpec((c_out, c_in), lambda i, s: (0, 0)),
            pl.BlockSpec((c_out, 1), lambda i, s: (0, 0)),
        ],
        out_specs=pl.BlockSpec((1, c_out, T), lambda i, s: (i, 0, s)),
        scratch_shapes=[pltpu.VMEM((c_in, T), jnp.bfloat16)],
        compiler_params=pltpu.CompilerParams(
            dimension_semantics=("parallel", "parallel"),
            vmem_limit_bytes=vmem_limit),
    )(high_flat, low_up_flat, w_all, b2d)

    out = out_flat.reshape(n, c_out, Hp, W)
    return out[:, :, :H, :] if Hp > H else out


# speedup vs baseline: 2.9412x; 2.6511x over previous
import functools
import math

import jax
import jax.numpy as jnp
import numpy as np
from jax.experimental import pallas as pl
from jax.experimental.pallas import tpu as pltpu


def _round_up(x: int, m: int) -> int:
    return ((x + m - 1) // m) * m


def _interp_matrix(out_size: int, in_size: int) -> np.ndarray:
    if in_size == 1:
        return np.ones((out_size, 1), np.float32)
    if out_size == 1:
        m = np.zeros((1, in_size), np.float32)
        m[0, 0] = 1.0
        return m
    dst = np.arange(out_size, dtype=np.float32)
    src = dst * np.float32((in_size - 1) / (out_size - 1))
    i0 = np.clip(np.floor(src).astype(np.int64), 0, in_size - 1)
    i1 = np.clip(i0 + 1, 0, in_size - 1)
    frac = (src - i0).astype(np.float32)
    rows = np.arange(out_size)
    m = np.zeros((out_size, in_size), np.float32)
    np.add.at(m, (rows, i0), 1.0 - frac)
    np.add.at(m, (rows, i1), frac)
    return m


def _upsample_kernel(c_lp, h, low_ref, mh_ref, mwt_ref, o_ref):
    lw = low_ref[0].astype(jnp.bfloat16)
    wi = jnp.dot(lw, mwt_ref[...], preferred_element_type=jnp.float32)
    wib = wi.astype(jnp.bfloat16)
    mh = mh_ref[...]
    for c in range(c_lp):
        zc = jnp.dot(mh, wib[c * h:(c + 1) * h],
                     preferred_element_type=jnp.float32)
        o_ref[0, c] = zc.astype(jnp.bfloat16)


def _conv_kernel(c_hp, high_ref, lowup_ref, w_ref, b_ref, o_ref, kbuf):
    _, c_out, r, wo = o_ref.shape
    t = r * wo
    c_lp = lowup_ref.shape[1]
    kbuf[0:c_hp, :] = high_ref[0].astype(jnp.bfloat16).reshape(c_hp, t)
    kbuf[c_hp:, :] = lowup_ref[0].reshape(c_lp, t)
    acc = jnp.dot(w_ref[...], kbuf[...], preferred_element_type=jnp.float32)
    acc = acc + b_ref[...]
    acc = acc * jnp.clip(acc + 3.0, 0.0, 6.0) * (1.0 / 6.0)
    o_ref[0] = acc.astype(o_ref.dtype).reshape(c_out, r, wo)


def kernel(low_res, high_res, weight, bias):
    n, c_lp, h, w = low_res.shape
    n2, c_hp, H, W = high_res.shape
    assert n == n2
    c_out = weight.shape[0]
    c_in = c_lp + c_hp
    out_dtype = high_res.dtype

    r_align = max(8, 128 // math.gcd(W, 128))
    R = r_align
    while R * 2 <= H and R * W < 8192:
        R *= 2
    if R > H:
        R = _round_up(H, r_align)
    Hp = _round_up(H, R)
    T = R * W

    m_h = _interp_matrix(H, h)
    m_wt = _interp_matrix(W, w).T
    if Hp > H:
        m_h = np.pad(m_h, ((0, Hp - H), (0, 0)))
        high_res = jnp.pad(high_res, ((0, 0), (0, 0), (0, Hp - H), (0, 0)))

    mh_b = m_h.astype(jnp.bfloat16)
    mwt_b = m_wt.astype(jnp.bfloat16)
    low2d = low_res.reshape(n, c_lp * h, w)

    vmem_limit = 48 * 1024 * 1024

    low_up = pl.pallas_call(
        functools.partial(_upsample_kernel, c_lp, h),
        out_shape=jax.ShapeDtypeStruct((n, c_lp, Hp, W), jnp.bfloat16),
        grid=(n,),
        in_specs=[
            pl.BlockSpec((1, c_lp * h, w), lambda i: (i, 0, 0)),
            pl.BlockSpec((Hp, h), lambda i: (0, 0)),
            pl.BlockSpec((w, W), lambda i: (0, 0)),
        ],
        out_specs=pl.BlockSpec((1, c_lp, Hp, W), lambda i: (i, 0, 0, 0)),
        compiler_params=pltpu.CompilerParams(
            dimension_semantics=("parallel",),
            vmem_limit_bytes=vmem_limit),
    )(low2d, mh_b, mwt_b)

    w_all = jnp.concatenate([weight[:, c_lp:], weight[:, :c_lp]],
                            axis=1).astype(jnp.bfloat16)
    b2d = bias.reshape(c_out, 1).astype(jnp.float32)

    out = pl.pallas_call(
        functools.partial(_conv_kernel, c_hp),
        out_shape=jax.ShapeDtypeStruct((n, c_out, Hp, W), out_dtype),
        grid=(n, Hp // R),
        in_specs=[
            pl.BlockSpec((1, c_hp, R, W), lambda i, s: (i, 0, s, 0)),
            pl.BlockSpec((1, c_lp, R, W), lambda i, s: (i, 0, s, 0)),
            pl.BlockSpec((c_out, c_in), lambda i, s: (0, 0)),
            pl.BlockSpec((c_out, 1), lambda i, s: (0, 0)),
        ],
        out_specs=pl.BlockSpec((1, c_out, R, W), lambda i, s: (i, 0, s, 0)),
        scratch_shapes=[pltpu.VMEM((c_in, T), jnp.bfloat16)],
        compiler_params=pltpu.CompilerParams(
            dimension_semantics=("parallel", "parallel"),
            vmem_limit_bytes=vmem_limit),
    )(high_res, low_up, w_all, b2d)

    return out[:, :, :H, :] if Hp > H else out


# speedup vs baseline: 3.1196x; 1.0606x over previous
import functools
import math

import jax
import jax.numpy as jnp
import numpy as np
from jax.experimental import pallas as pl
from jax.experimental.pallas import tpu as pltpu


def _round_up(x: int, m: int) -> int:
    return ((x + m - 1) // m) * m


def _interp_matrix(out_size: int, in_size: int) -> np.ndarray:
    if in_size == 1:
        return np.ones((out_size, 1), np.float32)
    if out_size == 1:
        m = np.zeros((1, in_size), np.float32)
        m[0, 0] = 1.0
        return m
    dst = np.arange(out_size, dtype=np.float32)
    src = dst * np.float32((in_size - 1) / (out_size - 1))
    i0 = np.clip(np.floor(src).astype(np.int64), 0, in_size - 1)
    i1 = np.clip(i0 + 1, 0, in_size - 1)
    frac = (src - i0).astype(np.float32)
    rows = np.arange(out_size)
    m = np.zeros((out_size, in_size), np.float32)
    np.add.at(m, (rows, i0), 1.0 - frac)
    np.add.at(m, (rows, i1), frac)
    return m


def _upsample_kernel(c_lp, h, low_ref, mh_ref, mwt_ref, o_ref):
    lw = low_ref[0].astype(jnp.bfloat16)
    wi = jnp.dot(lw, mwt_ref[...], preferred_element_type=jnp.float32)
    wib = wi.astype(jnp.bfloat16)
    mh = mh_ref[...]
    for c in range(c_lp):
        zc = jnp.dot(mh, wib[c * h:(c + 1) * h],
                     preferred_element_type=jnp.float32)
        o_ref[0, c] = zc.astype(jnp.bfloat16)


def _conv_kernel(c_hp, high_ref, lowup_ref, w_ref, b_ref, o_ref, kbuf):
    _, c_out, r, wo = o_ref.shape
    t = r * wo
    c_lp = lowup_ref.shape[1]
    kbuf[0:c_hp, :] = high_ref[0].astype(jnp.bfloat16).reshape(c_hp, t)
    kbuf[c_hp:, :] = lowup_ref[0].reshape(c_lp, t)
    acc = jnp.dot(w_ref[...], kbuf[...], preferred_element_type=jnp.float32)
    acc = acc + b_ref[...]
    acc = acc * jnp.clip(acc + 3.0, 0.0, 6.0) * (1.0 / 6.0)
    o_ref[0] = acc.astype(o_ref.dtype).reshape(c_out, r, wo)


def kernel(low_res, high_res, weight, bias):
    n, c_lp, h, w = low_res.shape
    n2, c_hp, H, W = high_res.shape
    assert n == n2
    c_out = weight.shape[0]
    c_in = c_lp + c_hp
    out_dtype = high_res.dtype

    r_align = max(8, 128 // math.gcd(W, 128))
    R = r_align
    while R * 2 <= H and R * W < 16384:
        R *= 2
    if R > H:
        R = _round_up(H, r_align)
    Hp = _round_up(H, R)
    T = R * W

    m_h = _interp_matrix(H, h)
    m_wt = _interp_matrix(W, w).T
    if Hp > H:
        m_h = np.pad(m_h, ((0, Hp - H), (0, 0)))
        high_res = jnp.pad(high_res, ((0, 0), (0, 0), (0, Hp - H), (0, 0)))

    mh_b = m_h.astype(jnp.bfloat16)
    mwt_b = m_wt.astype(jnp.bfloat16)
    low2d = low_res.reshape(n, c_lp * h, w)

    vmem_limit = 48 * 1024 * 1024

    low_up = pl.pallas_call(
        functools.partial(_upsample_kernel, c_lp, h),
        out_shape=jax.ShapeDtypeStruct((n, c_lp, Hp, W), jnp.bfloat16),
        grid=(n,),
        in_specs=[
            pl.BlockSpec((1, c_lp * h, w), lambda i: (i, 0, 0)),
            pl.BlockSpec((Hp, h), lambda i: (0, 0)),
            pl.BlockSpec((w, W), lambda i: (0, 0)),
        ],
        out_specs=pl.BlockSpec((1, c_lp, Hp, W), lambda i: (i, 0, 0, 0)),
        compiler_params=pltpu.CompilerParams(
            dimension_semantics=("parallel",),
            vmem_limit_bytes=vmem_limit),
    )(low2d, mh_b, mwt_b)

    w_all = jnp.concatenate([weight[:, c_lp:], weight[:, :c_lp]],
                            axis=1).astype(jnp.bfloat16)
    b2d = bias.reshape(c_out, 1).astype(jnp.float32)

    out = pl.pallas_call(
        functools.partial(_conv_kernel, c_hp),
        out_shape=jax.ShapeDtypeStruct((n, c_out, Hp, W), out_dtype),
        grid=(n, Hp // R),
        in_specs=[
            pl.BlockSpec((1, c_hp, R, W), lambda i, s: (i, 0, s, 0)),
            pl.BlockSpec((1, c_lp, R, W), lambda i, s: (i, 0, s, 0)),
            pl.BlockSpec((c_out, c_in), lambda i, s: (0, 0)),
            pl.BlockSpec((c_out, 1), lambda i, s: (0, 0)),
        ],
        out_specs=pl.BlockSpec((1, c_out, R, W), lambda i, s: (i, 0, s, 0)),
        scratch_shapes=[pltpu.VMEM((c_in, T), jnp.bfloat16)],
        compiler_params=pltpu.CompilerParams(
            dimension_semantics=("parallel", "parallel"),
            vmem_limit_bytes=vmem_limit),
    )(high_res, low_up, w_all, b2d)

    return out[:, :, :H, :] if Hp > H else out


# speedup vs baseline: 3.5066x; 1.1241x over previous
import functools
import math

import jax
import jax.numpy as jnp
import numpy as np
from jax.experimental import pallas as pl
from jax.experimental.pallas import tpu as pltpu


def _round_up(x: int, m: int) -> int:
    return ((x + m - 1) // m) * m


def _interp_matrix(out_size: int, in_size: int) -> np.ndarray:
    if in_size == 1:
        return np.ones((out_size, 1), np.float32)
    if out_size == 1:
        m = np.zeros((1, in_size), np.float32)
        m[0, 0] = 1.0
        return m
    dst = np.arange(out_size, dtype=np.float32)
    src = dst * np.float32((in_size - 1) / (out_size - 1))
    i0 = np.clip(np.floor(src).astype(np.int64), 0, in_size - 1)
    i1 = np.clip(i0 + 1, 0, in_size - 1)
    frac = (src - i0).astype(np.float32)
    rows = np.arange(out_size)
    m = np.zeros((out_size, in_size), np.float32)
    np.add.at(m, (rows, i0), 1.0 - frac)
    np.add.at(m, (rows, i1), frac)
    return m


def _ffm_kernel(c_lp, c_hp, h, low_ref, high_ref, mh_ref, mwt_ref, w_ref,
                b_ref, o_ref, kbuf):
    _, c_out, r, wo = o_ref.shape
    t = r * wo

    kbuf[0:c_hp, :] = high_ref[0].astype(jnp.bfloat16).reshape(c_hp, t)

    wi = jnp.dot(low_ref[0].astype(jnp.bfloat16), mwt_ref[...],
                 preferred_element_type=jnp.float32)
    wib = wi.astype(jnp.bfloat16)
    mh = mh_ref[...]
    for c in range(c_lp):
        zc = jnp.dot(mh, wib[c * h:(c + 1) * h],
                     preferred_element_type=jnp.float32)
        kbuf[c_hp + c:c_hp + c + 1, :] = zc.astype(jnp.bfloat16).reshape(1, t)

    acc = jnp.dot(w_ref[...], kbuf[...], preferred_element_type=jnp.float32)
    acc = acc + b_ref[...]
    acc = acc * jnp.clip(acc + 3.0, 0.0, 6.0) * (1.0 / 6.0)
    o_ref[0] = acc.astype(o_ref.dtype).reshape(c_out, r, wo)


def kernel(low_res, high_res, weight, bias):
    n, c_lp, h, w = low_res.shape
    n2, c_hp, H, W = high_res.shape
    assert n == n2
    c_out = weight.shape[0]
    c_in = c_lp + c_hp
    out_dtype = high_res.dtype

    r_align = max(8, 128 // math.gcd(W, 128))
    R = r_align
    while R * 2 <= H and R * W < 16384:
        R *= 2
    if R > H:
        R = _round_up(H, r_align)
    Hp = _round_up(H, R)
    T = R * W

    m_h = _interp_matrix(H, h)
    m_wt = _interp_matrix(W, w).T
    if Hp > H:
        m_h = np.pad(m_h, ((0, Hp - H), (0, 0)))
        high_res = jnp.pad(high_res, ((0, 0), (0, 0), (0, Hp - H), (0, 0)))

    mh_b = m_h.astype(jnp.bfloat16)
    mwt_b = m_wt.astype(jnp.bfloat16)
    low2d = low_res.reshape(n, c_lp * h, w)

    w_all = jnp.concatenate([weight[:, c_lp:], weight[:, :c_lp]],
                            axis=1).astype(jnp.bfloat16)
    b2d = bias.reshape(c_out, 1).astype(jnp.float32)

    out = pl.pallas_call(
        functools.partial(_ffm_kernel, c_lp, c_hp, h),
        out_shape=jax.ShapeDtypeStruct((n, c_out, Hp, W), out_dtype),
        grid=(n, Hp // R),
        in_specs=[
            pl.BlockSpec((1, c_lp * h, w), lambda i, s: (i, 0, 0)),
            pl.BlockSpec((1, c_hp, R, W), lambda i, s: (i, 0, s, 0)),
            pl.BlockSpec((R, h), lambda i, s: (s, 0)),
            pl.BlockSpec((w, W), lambda i, s: (0, 0)),
            pl.BlockSpec((c_out, c_in), lambda i, s: (0, 0)),
            pl.BlockSpec((c_out, 1), lambda i, s: (0, 0)),
        ],
        out_specs=pl.BlockSpec((1, c_out, R, W), lambda i, s: (i, 0, s, 0)),
        scratch_shapes=[pltpu.VMEM((c_in, T), jnp.bfloat16)],
        compiler_params=pltpu.CompilerParams(
            dimension_semantics=("parallel", "parallel"),
            vmem_limit_bytes=48 * 1024 * 1024),
    )(low2d, high_res, mh_b, mwt_b, w_all, b2d)

    return out[:, :, :H, :] if Hp > H else out


# speedup vs baseline: 3.5545x; 1.0137x over previous
import functools
import math

import jax
import jax.numpy as jnp
import numpy as np
from jax.experimental import pallas as pl
from jax.experimental.pallas import tpu as pltpu


def _round_up(x: int, m: int) -> int:
    return ((x + m - 1) // m) * m


def _interp_matrix(out_size: int, in_size: int) -> np.ndarray:
    if in_size == 1:
        return np.ones((out_size, 1), np.float32)
    if out_size == 1:
        m = np.zeros((1, in_size), np.float32)
        m[0, 0] = 1.0
        return m
    dst = np.arange(out_size, dtype=np.float32)
    src = dst * np.float32((in_size - 1) / (out_size - 1))
    i0 = np.clip(np.floor(src).astype(np.int64), 0, in_size - 1)
    i1 = np.clip(i0 + 1, 0, in_size - 1)
    frac = (src - i0).astype(np.float32)
    rows = np.arange(out_size)
    m = np.zeros((out_size, in_size), np.float32)
    np.add.at(m, (rows, i0), 1.0 - frac)
    np.add.at(m, (rows, i1), frac)
    return m


def _ffm_kernel(c_lp, c_hp, h, low_ref, high_ref, mh_ref, mwt_ref, w_ref,
                b_ref, o_ref, kbuf, zbuf):
    _, c_out, r, wo = o_ref.shape
    t = r * wo

    kbuf[0:c_hp, :] = high_ref[0].astype(jnp.bfloat16).reshape(c_hp, t)

    wi = jnp.dot(low_ref[0].astype(jnp.bfloat16), mwt_ref[...],
                 preferred_element_type=jnp.float32)
    wib = wi.astype(jnp.bfloat16)
    mh = mh_ref[...]
    for c in range(c_lp):
        zc = jnp.dot(mh, wib[c * h:(c + 1) * h],
                     preferred_element_type=jnp.float32)
        zbuf[c] = zc.astype(jnp.bfloat16)
    kbuf[c_hp:, :] = zbuf[...].reshape(c_lp, t)

    acc = jnp.dot(w_ref[...], kbuf[...], preferred_element_type=jnp.float32)
    acc = acc + b_ref[...]
    gate = jnp.clip(acc * (1.0 / 6.0) + 0.5, 0.0, 1.0)
    acc = acc * gate
    o_ref[0] = acc.astype(o_ref.dtype).reshape(c_out, r, wo)


def kernel(low_res, high_res, weight, bias):
    n, c_lp, h, w = low_res.shape
    n2, c_hp, H, W = high_res.shape
    assert n == n2
    c_out = weight.shape[0]
    c_in = c_lp + c_hp
    out_dtype = high_res.dtype

    r_align = max(8, 128 // math.gcd(W, 128))
    R = r_align
    while R * 2 <= H and R * W < 16384:
        R *= 2
    if R > H:
        R = _round_up(H, r_align)
    Hp = _round_up(H, R)
    T = R * W

    m_h = _interp_matrix(H, h)
    m_wt = _interp_matrix(W, w).T
    if Hp > H:
        m_h = np.pad(m_h, ((0, Hp - H), (0, 0)))
        high_res = jnp.pad(high_res, ((0, 0), (0, 0), (0, Hp - H), (0, 0)))

    mh_b = m_h.astype(jnp.bfloat16)
    mwt_b = m_wt.astype(jnp.bfloat16)
    low2d = low_res.reshape(n, c_lp * h, w)

    w_all = jnp.concatenate([weight[:, c_lp:], weight[:, :c_lp]],
                            axis=1).astype(jnp.bfloat16)
    b2d = bias.reshape(c_out, 1).astype(jnp.float32)

    out = pl.pallas_call(
        functools.partial(_ffm_kernel, c_lp, c_hp, h),
        out_shape=jax.ShapeDtypeStruct((n, c_out, Hp, W), out_dtype),
        grid=(n, Hp // R),
        in_specs=[
            pl.BlockSpec((1, c_lp * h, w), lambda i, s: (i, 0, 0)),
            pl.BlockSpec((1, c_hp, R, W), lambda i, s: (i, 0, s, 0)),
            pl.BlockSpec((R, h), lambda i, s: (s, 0)),
            pl.BlockSpec((w, W), lambda i, s: (0, 0)),
            pl.BlockSpec((c_out, c_in), lambda i, s: (0, 0)),
            pl.BlockSpec((c_out, 1), lambda i, s: (0, 0)),
        ],
        out_specs=pl.BlockSpec((1, c_out, R, W), lambda i, s: (i, 0, s, 0)),
        scratch_shapes=[pltpu.VMEM((c_in, T), jnp.bfloat16),
                        pltpu.VMEM((c_lp, R, W), jnp.bfloat16)],
        compiler_params=pltpu.CompilerParams(
            dimension_semantics=("parallel", "parallel"),
            vmem_limit_bytes=48 * 1024 * 1024),
    )(low2d, high_res, mh_b, mwt_b, w_all, b2d)

    return out[:, :, :H, :] if Hp > H else out
